# baseline (speedup 1.0000x reference)
import jax
import jax.numpy as jnp
from jax.experimental import pallas as pl
from jax.experimental.pallas import tpu as pltpu

N = 10000
E = 160000
EDGE_BLK = 4000
M_BLK = 1000


def _agg_body(idx_ref, x_ref, out_ref):
    @pl.when(pl.program_id(0) == 0)
    def _zero():
        out_ref[...] = jnp.zeros_like(out_ref)

    def step(e, carry):
        s = idx_ref[0, 0, e]
        d = idx_ref[0, 1, e]
        out_ref[pl.ds(d, 1), :] += x_ref[pl.ds(s, 1), :]
        return carry

    jax.lax.fori_loop(0, EDGE_BLK, step, 0)


def _segment_sum(x, idx_blocked):
    n, d = x.shape
    grid = (E // EDGE_BLK,)
    return pl.pallas_call(
        _agg_body,
        grid=grid,
        in_specs=[
            pl.BlockSpec((1, 2, EDGE_BLK), lambda i: (i, 0, 0), memory_space=pltpu.SMEM),
            pl.BlockSpec((n, d), lambda i: (0, 0)),
        ],
        out_specs=pl.BlockSpec((n, d), lambda i: (0, 0)),
        out_shape=jax.ShapeDtypeStruct((n, d), jnp.float32),
    )(idx_blocked, x)


def _mlp_body(x_ref, a_ref, wa_ref, ba_ref, wb_ref, bb_ref, o_ref):
    h = x_ref[...] + a_ref[...]
    h = jnp.dot(h, wa_ref[...], preferred_element_type=jnp.float32) + ba_ref[...]
    h = jnp.maximum(h, 0.0)
    h = jnp.dot(h, wb_ref[...], preferred_element_type=jnp.float32) + bb_ref[...]
    o_ref[...] = jnp.maximum(h, 0.0)


def _mlp(x, agg, Wa, ba, Wb, bb):
    n, d_in = x.shape
    d_hid = Wa.shape[1]
    d_out = Wb.shape[1]
    ba2 = ba.reshape(1, -1)
    bb2 = bb.reshape(1, -1)
    grid = (n // M_BLK,)
    return pl.pallas_call(
        _mlp_body,
        grid=grid,
        in_specs=[
            pl.BlockSpec((M_BLK, d_in), lambda i: (i, 0)),
            pl.BlockSpec((M_BLK, d_in), lambda i: (i, 0)),
            pl.BlockSpec((d_in, d_hid), lambda i: (0, 0)),
            pl.BlockSpec((1, d_hid), lambda i: (0, 0)),
            pl.BlockSpec((d_hid, d_out), lambda i: (0, 0)),
            pl.BlockSpec((1, d_out), lambda i: (0, 0)),
        ],
        out_specs=pl.BlockSpec((M_BLK, d_out), lambda i: (i, 0)),
        out_shape=jax.ShapeDtypeStruct((n, d_out), jnp.float32),
    )(x, agg, Wa, ba2, Wb, bb2)


def kernel(x, edge_index, W1a, b1a, W1b, b1b, W2a, b2a, W2b, b2b):
    idx = edge_index.astype(jnp.int32)
    idx = idx.reshape(2, E // EDGE_BLK, EDGE_BLK).transpose(1, 0, 2)
    agg1 = _segment_sum(x, idx)
    h1 = _mlp(x, agg1, W1a, b1a, W1b, b1b)
    agg2 = _segment_sum(h1, idx)
    h2 = _mlp(h1, agg2, W2a, b2a, W2b, b2b)
    return h2


# speedup vs baseline: 1.8358x; 1.8358x over previous
import functools

import jax
import jax.numpy as jnp
from jax import lax
from jax.experimental import pallas as pl
from jax.experimental.pallas import tpu as pltpu
from jax.experimental.pallas import tpu_sc as plsc

N = 10000
E = 160000
D_IN = 256
D_HID = 512

_NC = 2
_NS = 16
_CH = 128
_NCHUNK = 40
_EPW = _NCHUNK * _CH
_E_PAD = _NC * _NS * _EPW
_N_PAD = 10112
_RPT = _N_PAD // _NS
_M_BLK = 2528


def _agg_body(table_hbm, src_hbm, dst_hbm, z_hbm, out_hbm,
              src_v, dst_v, rows_v, aggm, sem):
    c = lax.axis_index("c")
    s = lax.axis_index("s")
    wid = c * _NS + s
    pltpu.sync_copy(z_hbm, aggm.at[pl.ds(s * _RPT, _RPT)])
    plsc.subcore_barrier()

    def chunk(k_, carry):
        pltpu.sync_copy(src_hbm.at[wid, k_], src_v)
        pltpu.sync_copy(dst_hbm.at[wid, k_], dst_v)
        pltpu.async_copy(table_hbm.at[src_v], rows_v, sem).wait()
        pltpu.sync_copy(rows_v, aggm.at[dst_v], add=True)
        return carry

    lax.fori_loop(0, _NCHUNK, chunk, 0)
    plsc.subcore_barrier()
    pltpu.sync_copy(aggm.at[pl.ds(s * _RPT, _RPT)],
                    out_hbm.at[c, pl.ds(s * _RPT, _RPT)])


def _agg_sc(table, src2, dst2, z):
    mesh = plsc.VectorSubcoreMesh(core_axis_name="c", subcore_axis_name="s")
    f = pl.kernel(
        _agg_body,
        mesh=mesh,
        out_type=jax.ShapeDtypeStruct((_NC, _N_PAD, 128), jnp.float32),
        scratch_types=[
            pltpu.VMEM((_CH,), jnp.int32),
            pltpu.VMEM((_CH,), jnp.int32),
            pltpu.VMEM((_CH, 128), jnp.float32),
            pltpu.VMEM_SHARED((_N_PAD, 128), jnp.float32),
            pltpu.SemaphoreType.DMA,
        ],
    )
    return f(table, src2, dst2, z)


def _mlp1_body(x2_ref, p0_ref, p1_ref, wa_ref, ba_ref, wb_ref, bb_ref, o_ref):
    xin = jnp.concatenate([x2_ref[0], x2_ref[1]], axis=1)
    agg = jnp.concatenate([p0_ref[0] + p0_ref[1], p1_ref[0] + p1_ref[1]], axis=1)
    h = xin + agg
    h = jnp.dot(h, wa_ref[...], preferred_element_type=jnp.float32) + ba_ref[...]
    h = jnp.maximum(h, 0.0)
    h = jnp.dot(h, wb_ref[...], preferred_element_type=jnp.float32) + bb_ref[...]
    h = jnp.maximum(h, 0.0)
    for c in range(4):
        o_ref[c] = h[:, c * 128:(c + 1) * 128]


def _mlp1(x2, p0, p1, Wa, ba, Wb, bb):
    grid = (_N_PAD // _M_BLK,)
    return pl.pallas_call(
        _mlp1_body,
        grid=grid,
        in_specs=[
            pl.BlockSpec((2, _M_BLK, 128), lambda i: (0, i, 0)),
            pl.BlockSpec((2, _M_BLK, 128), lambda i: (0, i, 0)),
            pl.BlockSpec((2, _M_BLK, 128), lambda i: (0, i, 0)),
            pl.BlockSpec((D_IN, D_HID), lambda i: (0, 0)),
            pl.BlockSpec((1, D_HID), lambda i: (0, 0)),
            pl.BlockSpec((D_HID, D_HID), lambda i: (0, 0)),
            pl.BlockSpec((1, D_HID), lambda i: (0, 0)),
        ],
        out_specs=pl.BlockSpec((4, _M_BLK, 128), lambda i: (0, i, 0)),
        out_shape=jax.ShapeDtypeStruct((4, _N_PAD, 128), jnp.float32),
    )(x2, p0, p1, Wa, ba.reshape(1, -1), Wb, bb.reshape(1, -1))


def _mlp2_body(h2_ref, q0_ref, q1_ref, q2_ref, q3_ref,
               wa_ref, ba_ref, wb_ref, bb_ref, o_ref):
    qs = [q0_ref, q1_ref, q2_ref, q3_ref]
    xin = jnp.concatenate([h2_ref[c] for c in range(4)], axis=1)
    agg = jnp.concatenate([qs[c][0] + qs[c][1] for c in range(4)], axis=1)
    h = xin + agg
    h = jnp.dot(h, wa_ref[...], preferred_element_type=jnp.float32) + ba_ref[...]
    h = jnp.maximum(h, 0.0)
    h = jnp.dot(h, wb_ref[...], preferred_element_type=jnp.float32) + bb_ref[...]
    o_ref[...] = jnp.maximum(h, 0.0)


def _mlp2(h2, qs, Wa, ba, Wb, bb):
    grid = (_N_PAD // _M_BLK,)
    return pl.pallas_call(
        _mlp2_body,
        grid=grid,
        in_specs=[
            pl.BlockSpec((4, _M_BLK, 128), lambda i: (0, i, 0)),
            pl.BlockSpec((2, _M_BLK, 128), lambda i: (0, i, 0)),
            pl.BlockSpec((2, _M_BLK, 128), lambda i: (0, i, 0)),
            pl.BlockSpec((2, _M_BLK, 128), lambda i: (0, i, 0)),
            pl.BlockSpec((2, _M_BLK, 128), lambda i: (0, i, 0)),
            pl.BlockSpec((D_HID, D_HID), lambda i: (0, 0)),
            pl.BlockSpec((1, D_HID), lambda i: (0, 0)),
            pl.BlockSpec((D_HID, D_HID), lambda i: (0, 0)),
            pl.BlockSpec((1, D_HID), lambda i: (0, 0)),
        ],
        out_specs=pl.BlockSpec((_M_BLK, D_HID), lambda i: (i, 0)),
        out_shape=jax.ShapeDtypeStruct((_N_PAD, D_HID), jnp.float32),
    )(h2, qs[0], qs[1], qs[2], qs[3], Wa, ba.reshape(1, -1), Wb, bb.reshape(1, -1))


def kernel(x, edge_index, W1a, b1a, W1b, b1b, W2a, b2a, W2b, b2b):
    idx = edge_index.astype(jnp.int32)
    pad = _E_PAD - E
    src2 = jnp.concatenate([idx[0], jnp.zeros((pad,), jnp.int32)]).reshape(
        _NC * _NS, _NCHUNK, _CH)
    dst2 = jnp.concatenate([idx[1], jnp.full((pad,), N, jnp.int32)]).reshape(
        _NC * _NS, _NCHUNK, _CH)
    z = jnp.zeros((_RPT, 128), jnp.float32)

    xp = jnp.pad(x, ((0, _N_PAD - N), (0, 0)))
    x2 = xp.reshape(_N_PAD, 2, 128).transpose(1, 0, 2)

    p0 = _agg_sc(x2[0], src2, dst2, z)
    p1 = _agg_sc(x2[1], src2, dst2, z)
    h2 = _mlp1(x2, p0, p1, W1a, b1a, W1b, b1b)

    q = [_agg_sc(h2[c], src2, dst2, z) for c in range(4)]
    out = _mlp2(h2, q, W2a, b2a, W2b, b2b)
    return out[:N]


# speedup vs baseline: 2.2286x; 1.2140x over previous
import jax
import jax.numpy as jnp
from jax import lax
from jax.experimental import pallas as pl
from jax.experimental.pallas import tpu as pltpu
from jax.experimental.pallas import tpu_sc as plsc

N = 10000
E = 160000
D_IN = 256
D_HID = 512

_NC = 2
_NS = 16
_CH = 128
_NCHUNK = 40
_EPW = _NCHUNK * _CH
_E_PAD = _NC * _NS * _EPW
_N_PAD = 10112
_RPT = _N_PAD // _NS
_M_BLK = 2528


def _make_agg_body(nchk):
    def body(*refs):
        tables = refs[:nchk]
        src_hbm, dst_hbm, z_hbm, out_hbm = refs[nchk:nchk + 4]
        src_v, dst_v, rows_a, rows_b, aggm, sem_a, sem_b = refs[nchk + 4:]
        c = lax.axis_index("c")
        s = lax.axis_index("s")
        wid = c * _NS + s
        my = pl.ds(s * _RPT, _RPT)
        pltpu.sync_copy(src_hbm.at[wid], src_v)
        pltpu.sync_copy(dst_hbm.at[wid], dst_v)
        pltpu.sync_copy(z_hbm, aggm.at[my])
        plsc.subcore_barrier()
        for cc in range(nchk):
            tab = tables[cc]
            pltpu.async_copy(tab.at[src_v.at[0]], rows_a, sem_a)
            pltpu.async_copy(tab.at[src_v.at[1]], rows_b, sem_b)

            def pair(it, carry):
                k = 2 * it
                pltpu.make_async_copy(tab.at[src_v.at[k]], rows_a, sem_a).wait()
                pltpu.sync_copy(rows_a, aggm.at[dst_v.at[k]], add=True)
                pltpu.async_copy(tab.at[src_v.at[k + 2]], rows_a, sem_a)
                pltpu.make_async_copy(tab.at[src_v.at[k + 1]], rows_b, sem_b).wait()
                pltpu.sync_copy(rows_b, aggm.at[dst_v.at[k + 1]], add=True)
                pltpu.async_copy(tab.at[src_v.at[k + 3]], rows_b, sem_b)
                return carry

            lax.fori_loop(0, _NCHUNK // 2 - 1, pair, 0)
            kl = _NCHUNK - 2
            pltpu.make_async_copy(tab.at[src_v.at[kl]], rows_a, sem_a).wait()
            pltpu.sync_copy(rows_a, aggm.at[dst_v.at[kl]], add=True)
            pltpu.make_async_copy(tab.at[src_v.at[kl + 1]], rows_b, sem_b).wait()
            pltpu.sync_copy(rows_b, aggm.at[dst_v.at[kl + 1]], add=True)

            plsc.subcore_barrier()
            pltpu.sync_copy(aggm.at[my], out_hbm.at[cc, c, my])
            if cc + 1 < nchk:
                pltpu.sync_copy(z_hbm, aggm.at[my])
                plsc.subcore_barrier()
    return body


def _agg_sc(tables, src2, dst2, z):
    nchk = len(tables)
    mesh = plsc.VectorSubcoreMesh(core_axis_name="c", subcore_axis_name="s")
    f = pl.kernel(
        _make_agg_body(nchk),
        mesh=mesh,
        out_type=jax.ShapeDtypeStruct((nchk, _NC, _N_PAD, 128), jnp.float32),
        scratch_types=[
            pltpu.VMEM((_NCHUNK, _CH), jnp.int32),
            pltpu.VMEM((_NCHUNK, _CH), jnp.int32),
            pltpu.VMEM((_CH, 128), jnp.float32),
            pltpu.VMEM((_CH, 128), jnp.float32),
            pltpu.VMEM_SHARED((_N_PAD, 128), jnp.float32),
            pltpu.SemaphoreType.DMA,
            pltpu.SemaphoreType.DMA,
        ],
    )
    return f(*tables, src2, dst2, z)


def _mlp1_body(x2_ref, p_ref, wa_ref, ba_ref, wb_ref, bb_ref, o_ref):
    xin = jnp.concatenate([x2_ref[0], x2_ref[1]], axis=1)
    agg = jnp.concatenate([p_ref[0, 0] + p_ref[0, 1],
                           p_ref[1, 0] + p_ref[1, 1]], axis=1)
    h = xin + agg
    h = jnp.dot(h, wa_ref[...], preferred_element_type=jnp.float32) + ba_ref[...]
    h = jnp.maximum(h, 0.0)
    h = jnp.dot(h, wb_ref[...], preferred_element_type=jnp.float32) + bb_ref[...]
    h = jnp.maximum(h, 0.0)
    for c in range(4):
        o_ref[c] = h[:, c * 128:(c + 1) * 128]


def _mlp1(x2, p, Wa, ba, Wb, bb):
    grid = (_N_PAD // _M_BLK,)
    return pl.pallas_call(
        _mlp1_body,
        grid=grid,
        in_specs=[
            pl.BlockSpec((2, _M_BLK, 128), lambda i: (0, i, 0)),
            pl.BlockSpec((2, 2, _M_BLK, 128), lambda i: (0, 0, i, 0)),
            pl.BlockSpec((D_IN, D_HID), lambda i: (0, 0)),
            pl.BlockSpec((1, D_HID), lambda i: (0, 0)),
            pl.BlockSpec((D_HID, D_HID), lambda i: (0, 0)),
            pl.BlockSpec((1, D_HID), lambda i: (0, 0)),
        ],
        out_specs=pl.BlockSpec((4, _M_BLK, 128), lambda i: (0, i, 0)),
        out_shape=jax.ShapeDtypeStruct((4, _N_PAD, 128), jnp.float32),
    )(x2, p, Wa, ba.reshape(1, -1), Wb, bb.reshape(1, -1))


def _mlp2_body(h2_ref, q_ref, wa_ref, ba_ref, wb_ref, bb_ref, o_ref):
    xin = jnp.concatenate([h2_ref[c] for c in range(4)], axis=1)
    agg = jnp.concatenate([q_ref[c, 0] + q_ref[c, 1] for c in range(4)], axis=1)
    h = xin + agg
    h = jnp.dot(h, wa_ref[...], preferred_element_type=jnp.float32) + ba_ref[...]
    h = jnp.maximum(h, 0.0)
    h = jnp.dot(h, wb_ref[...], preferred_element_type=jnp.float32) + bb_ref[...]
    o_ref[...] = jnp.maximum(h, 0.0)


def _mlp2(h2, q, Wa, ba, Wb, bb):
    grid = (_N_PAD // _M_BLK,)
    return pl.pallas_call(
        _mlp2_body,
        grid=grid,
        in_specs=[
            pl.BlockSpec((4, _M_BLK, 128), lambda i: (0, i, 0)),
            pl.BlockSpec((4, 2, _M_BLK, 128), lambda i: (0, 0, i, 0)),
            pl.BlockSpec((D_HID, D_HID), lambda i: (0, 0)),
            pl.BlockSpec((1, D_HID), lambda i: (0, 0)),
            pl.BlockSpec((D_HID, D_HID), lambda i: (0, 0)),
            pl.BlockSpec((1, D_HID), lambda i: (0, 0)),
        ],
        out_specs=pl.BlockSpec((_M_BLK, D_HID), lambda i: (i, 0)),
        out_shape=jax.ShapeDtypeStruct((_N_PAD, D_HID), jnp.float32),
    )(h2, q, Wa, ba.reshape(1, -1), Wb, bb.reshape(1, -1))


def kernel(x, edge_index, W1a, b1a, W1b, b1b, W2a, b2a, W2b, b2b):
    idx = edge_index.astype(jnp.int32)
    pad = _E_PAD - E
    src2 = jnp.concatenate([idx[0], jnp.zeros((pad,), jnp.int32)]).reshape(
        _NC * _NS, _NCHUNK, _CH)
    dst2 = jnp.concatenate([idx[1], jnp.full((pad,), N, jnp.int32)]).reshape(
        _NC * _NS, _NCHUNK, _CH)
    z = jnp.zeros((_RPT, 128), jnp.float32)

    xp = jnp.pad(x, ((0, _N_PAD - N), (0, 0)))
    x2 = xp.reshape(_N_PAD, 2, 128).transpose(1, 0, 2)

    p = _agg_sc([x2[0], x2[1]], src2, dst2, z)
    h2 = _mlp1(x2, p, W1a, b1a, W1b, b1b)
    q = _agg_sc([h2[0], h2[1], h2[2], h2[3]], src2, dst2, z)
    out = _mlp2(h2, q, W2a, b2a, W2b, b2b)
    return out[:N]
